# Initial kernel scaffold; baseline (speedup 1.0000x reference)
import jax
import jax.numpy as jnp
from jax.experimental import pallas as pl


def kernel(x, codebooks):
    raise NotImplementedError("write your pallas kernel here")


# speedup vs baseline: 1.4259x; 1.4259x over previous
import jax
import jax.numpy as jnp
from jax import lax
from jax.experimental import pallas as pl
from jax.experimental.pallas import tpu as pltpu

_NQ = 8
_K = 1024
_D = 64
_TB = 512


def _rvq_body(x_ref, x2_ref, cb_ref, cbt_ref, cb2_ref, out_ref, loss_ref):
    t = pl.program_id(1)
    x0 = x_ref[0]
    x2 = x2_ref[0]
    r = x0
    acc = jnp.zeros((1, 128), jnp.float32)
    kio = lax.broadcasted_iota(jnp.int32, (_K, _TB), 0)
    for i in range(_NQ):
        cb = cb_ref[i]
        m = lax.dot_general(cb, r, (((1,), (0,)), ((), ())),
                            preferred_element_type=jnp.float32)
        dist = (x2 - 2.0 * m) + cb2_ref[i]
        mn = jnp.min(dist, axis=0, keepdims=True)
        idx = jnp.min(jnp.where(dist == mn, kio, _K), axis=0,
                      keepdims=True)
        oh = (kio == idx).astype(jnp.float32)
        q = lax.dot_general(cbt_ref[i], oh, (((1,), (0,)), ((), ())),
                            preferred_element_type=jnp.float32,
                            precision=lax.Precision.HIGHEST)
        r = r - q
        acc = acc + jnp.sum(r * r)

    out_ref[0] = x0 - r

    @pl.when(t == 0)
    def _init():
        loss_ref[...] = jnp.zeros_like(loss_ref)

    loss_ref[...] += acc[None]


def kernel(x, codebooks):
    B, D, T = x.shape
    x2 = jnp.sum(x * x, axis=1).reshape(B, 1, T)
    cb2 = jnp.sum(codebooks * codebooks, axis=-1)[..., None]
    cbt = jnp.swapaxes(codebooks, 1, 2)
    grid = (B, T // _TB)
    out, lossv = pl.pallas_call(
        _rvq_body,
        grid=grid,
        in_specs=[
            pl.BlockSpec((1, D, _TB), lambda b, t: (b, 0, t)),
            pl.BlockSpec((1, 1, _TB), lambda b, t: (b, 0, t)),
            pl.BlockSpec((_NQ, _K, D), lambda b, t: (0, 0, 0)),
            pl.BlockSpec((_NQ, D, _K), lambda b, t: (0, 0, 0)),
            pl.BlockSpec((_NQ, _K, 1), lambda b, t: (0, 0, 0)),
        ],
        out_specs=[
            pl.BlockSpec((1, D, _TB), lambda b, t: (b, 0, t)),
            pl.BlockSpec((1, 1, 128), lambda b, t: (b, 0, 0)),
        ],
        out_shape=[
            jax.ShapeDtypeStruct((B, D, T), jnp.float32),
            jax.ShapeDtypeStruct((B, 1, 128), jnp.float32),
        ],
        compiler_params=pltpu.CompilerParams(
            dimension_semantics=("arbitrary", "arbitrary")),
    )(x, x2, codebooks, cbt, cb2)
    losses = lossv[:, 0, 0] * (1.0 / (D * T))
    return (out, losses)


# speedup vs baseline: 2.6921x; 1.8880x over previous
import jax
import jax.numpy as jnp
from jax import lax
from jax.experimental import pallas as pl
from jax.experimental.pallas import tpu as pltpu

_NQ = 8
_K = 1024
_D = 64
_TB = 512


def _rvq_body(x_ref, x2_ref, cb_ref, cbt3_ref, cb2_ref, out_ref, loss_ref):
    t = pl.program_id(1)
    x0 = x_ref[0]
    x2 = x2_ref[0]
    r = x0
    acc = jnp.zeros((1, 128), jnp.float32)
    kio = lax.broadcasted_iota(jnp.int32, (_K, _TB), 0)
    for i in range(_NQ):
        cb = cb_ref[i]
        m = lax.dot_general(cb, r, (((1,), (0,)), ((), ())),
                            preferred_element_type=jnp.float32)
        dist = (x2 - 2.0 * m) + cb2_ref[i]
        mn = jnp.min(dist, axis=0, keepdims=True)
        idx = jnp.min(jnp.where(dist == mn, kio, _K), axis=0,
                      keepdims=True)
        oh = (kio == idx).astype(jnp.bfloat16)
        q3 = lax.dot_general(cbt3_ref[i], oh, (((1,), (0,)), ((), ())),
                             preferred_element_type=jnp.float32)
        q = (q3[0:_D] + q3[_D:2 * _D]) + q3[2 * _D:3 * _D]
        r = r - q
        acc = acc + jnp.sum(r * r)

    out_ref[0] = x0 - r

    @pl.when(t == 0)
    def _init():
        loss_ref[...] = jnp.zeros_like(loss_ref)

    loss_ref[...] += acc[None]


def kernel(x, codebooks):
    B, D, T = x.shape
    x2 = jnp.sum(x * x, axis=1).reshape(B, 1, T)
    cb2 = jnp.sum(codebooks * codebooks, axis=-1)[..., None]
    cbt = jnp.swapaxes(codebooks, 1, 2)
    top16 = jnp.uint32(0xFFFF0000)
    cbt_hi = lax.bitcast_convert_type(
        lax.bitcast_convert_type(cbt, jnp.uint32) & top16, jnp.float32)
    rem = cbt - cbt_hi
    cbt_mid = lax.bitcast_convert_type(
        lax.bitcast_convert_type(rem, jnp.uint32) & top16, jnp.float32)
    cbt_lo = rem - cbt_mid
    cbt3 = jnp.concatenate([cbt_hi, cbt_mid, cbt_lo],
                           axis=1).astype(jnp.bfloat16)
    grid = (B, T // _TB)
    out, lossv = pl.pallas_call(
        _rvq_body,
        grid=grid,
        in_specs=[
            pl.BlockSpec((1, D, _TB), lambda b, t: (b, 0, t)),
            pl.BlockSpec((1, 1, _TB), lambda b, t: (b, 0, t)),
            pl.BlockSpec((_NQ, _K, D), lambda b, t: (0, 0, 0)),
            pl.BlockSpec((_NQ, 3 * D, _K), lambda b, t: (0, 0, 0)),
            pl.BlockSpec((_NQ, _K, 1), lambda b, t: (0, 0, 0)),
        ],
        out_specs=[
            pl.BlockSpec((1, D, _TB), lambda b, t: (b, 0, t)),
            pl.BlockSpec((1, 1, 128), lambda b, t: (b, 0, 0)),
        ],
        out_shape=[
            jax.ShapeDtypeStruct((B, D, T), jnp.float32),
            jax.ShapeDtypeStruct((B, 1, 128), jnp.float32),
        ],
        compiler_params=pltpu.CompilerParams(
            dimension_semantics=("arbitrary", "arbitrary")),
    )(x, x2, codebooks, cbt3, cb2)
    losses = lossv[:, 0, 0] * (1.0 / (D * T))
    return (out, losses)


# speedup vs baseline: 2.7707x; 1.0292x over previous
import jax
import jax.numpy as jnp
from jax import lax
from jax.experimental import pallas as pl
from jax.experimental.pallas import tpu as pltpu

_NQ = 8
_K = 1024
_D = 64
_TB = 512


def _split3(v):
    top16 = jnp.uint32(0xFFFF0000)
    hi = lax.bitcast_convert_type(
        lax.bitcast_convert_type(v, jnp.uint32) & top16, jnp.float32)
    rem = v - hi
    mid = lax.bitcast_convert_type(
        lax.bitcast_convert_type(rem, jnp.uint32) & top16, jnp.float32)
    lo = rem - mid
    return hi, mid, lo


def _rvq_body(x_ref, cba_ref, cbt3_ref, out_ref, loss_ref):
    t = pl.program_id(1)
    x0 = x_ref[0]
    r = x0
    acc = jnp.zeros((1, 128), jnp.float32)
    kio = lax.broadcasted_iota(jnp.int32, (_K, _TB), 0)
    ones3 = jnp.ones((3, _TB), jnp.float32)
    for i in range(_NQ):
        x2 = jnp.sum(r * r, axis=0, keepdims=True)
        r_aug = jnp.concatenate([r, x2, ones3],
                                axis=0).astype(jnp.bfloat16)
        dist = lax.dot_general(cba_ref[i], r_aug, (((1,), (0,)), ((), ())),
                               preferred_element_type=jnp.float32)
        mn = jnp.min(dist, axis=0, keepdims=True)
        idx = jnp.min(jnp.where(dist == mn, kio, _K), axis=0,
                      keepdims=True)
        oh = (kio == idx).astype(jnp.bfloat16)
        q3 = lax.dot_general(cbt3_ref[i], oh, (((1,), (0,)), ((), ())),
                             preferred_element_type=jnp.float32)
        q = (q3[0:_D] + q3[_D:2 * _D]) + q3[2 * _D:3 * _D]
        r = r - q
        acc = acc + jnp.sum(mn)

    out_ref[0] = x0 - r

    @pl.when(t == 0)
    def _init():
        loss_ref[...] = jnp.zeros_like(loss_ref)

    loss_ref[...] += acc[None]


def kernel(x, codebooks):
    B, D, T = x.shape
    cb2 = jnp.sum(codebooks * codebooks, axis=-1)
    c2h, c2m, c2l = _split3(cb2)
    cba = jnp.concatenate(
        [-2.0 * codebooks,
         jnp.ones((_NQ, _K, 1), jnp.float32),
         c2h[..., None], c2m[..., None], c2l[..., None]],
        axis=-1).astype(jnp.bfloat16)
    cbt = jnp.swapaxes(codebooks, 1, 2)
    h, m_, l = _split3(cbt)
    cbt3 = jnp.concatenate([h, m_, l], axis=1).astype(jnp.bfloat16)
    grid = (B, T // _TB)
    out, lossv = pl.pallas_call(
        _rvq_body,
        grid=grid,
        in_specs=[
            pl.BlockSpec((1, D, _TB), lambda b, t: (b, 0, t)),
            pl.BlockSpec((_NQ, _K, D + 4), lambda b, t: (0, 0, 0)),
            pl.BlockSpec((_NQ, 3 * D, _K), lambda b, t: (0, 0, 0)),
        ],
        out_specs=[
            pl.BlockSpec((1, D, _TB), lambda b, t: (b, 0, t)),
            pl.BlockSpec((1, 1, 128), lambda b, t: (b, 0, 0)),
        ],
        out_shape=[
            jax.ShapeDtypeStruct((B, D, T), jnp.float32),
            jax.ShapeDtypeStruct((B, 1, 128), jnp.float32),
        ],
        compiler_params=pltpu.CompilerParams(
            dimension_semantics=("arbitrary", "arbitrary")),
    )(x, cba, cbt3)
    losses = lossv[:, 0, 0] * (1.0 / (D * T))
    return (out, losses)


# speedup vs baseline: 2.8166x; 1.0166x over previous
import jax
import jax.numpy as jnp
from jax import lax
from jax.experimental import pallas as pl
from jax.experimental.pallas import tpu as pltpu

_NQ = 8
_K = 1024
_D = 64
_TB = 512


def _split3(v):
    top16 = jnp.uint32(0xFFFF0000)
    hi = lax.bitcast_convert_type(
        lax.bitcast_convert_type(v, jnp.uint32) & top16, jnp.float32)
    rem = v - hi
    mid = lax.bitcast_convert_type(
        lax.bitcast_convert_type(rem, jnp.uint32) & top16, jnp.float32)
    lo = rem - mid
    return hi, mid, lo


def _rvq_body(x_ref, cba_ref, cbt4_ref, out_ref, loss_ref):
    t = pl.program_id(1)
    x0 = x_ref[0]
    r = x0
    acc = jnp.zeros((1, 128), jnp.float32)
    ones3 = jnp.ones((3, _TB), jnp.float32)
    x2 = jnp.sum(x0 * x0, axis=0, keepdims=True)
    for i in range(_NQ):
        r_aug = jnp.concatenate([r, x2, ones3],
                                axis=0).astype(jnp.bfloat16)
        dist = lax.dot_general(cba_ref[i], r_aug, (((1,), (0,)), ((), ())),
                               preferred_element_type=jnp.float32)
        mn = jnp.min(dist, axis=0, keepdims=True)
        ohm = dist == mn
        oh = ohm.astype(jnp.bfloat16)
        q4 = lax.dot_general(cbt4_ref[i], oh, (((1,), (0,)), ((), ())),
                             preferred_element_type=jnp.float32)
        cnt = q4[3 * _D:3 * _D + 1]

        def _fix_ties(ohm=ohm, i=i):
            kio = lax.broadcasted_iota(jnp.int32, (_K, _TB), 0)
            idx = jnp.min(jnp.where(ohm, kio, _K), axis=0, keepdims=True)
            oh1 = (kio == idx).astype(jnp.bfloat16)
            return lax.dot_general(cbt4_ref[i][0:3 * _D], oh1,
                                   (((1,), (0,)), ((), ())),
                                   preferred_element_type=jnp.float32)

        q3 = lax.cond(jnp.max(cnt) > 1.5, _fix_ties, lambda: q4[0:3 * _D])
        q = (q3[0:_D] + q3[_D:2 * _D]) + q3[2 * _D:3 * _D]
        r = r - q
        acc = acc + jnp.sum(mn)
        x2 = mn

    out_ref[0] = x0 - r

    @pl.when(t == 0)
    def _init():
        loss_ref[...] = jnp.zeros_like(loss_ref)

    loss_ref[...] += acc[None]


def kernel(x, codebooks):
    B, D, T = x.shape
    cb2 = jnp.sum(codebooks * codebooks, axis=-1)
    c2h, c2m, c2l = _split3(cb2)
    cba = jnp.concatenate(
        [-2.0 * codebooks,
         jnp.ones((_NQ, _K, 1), jnp.float32),
         c2h[..., None], c2m[..., None], c2l[..., None]],
        axis=-1).astype(jnp.bfloat16)
    cbt = jnp.swapaxes(codebooks, 1, 2)
    h, m_, l = _split3(cbt)
    cbt4 = jnp.concatenate(
        [h, m_, l, jnp.ones((_NQ, 1, _K), jnp.float32)],
        axis=1).astype(jnp.bfloat16)
    grid = (B, T // _TB)
    out, lossv = pl.pallas_call(
        _rvq_body,
        grid=grid,
        in_specs=[
            pl.BlockSpec((1, D, _TB), lambda b, t: (b, 0, t)),
            pl.BlockSpec((_NQ, _K, D + 4), lambda b, t: (0, 0, 0)),
            pl.BlockSpec((_NQ, 3 * D + 1, _K), lambda b, t: (0, 0, 0)),
        ],
        out_specs=[
            pl.BlockSpec((1, D, _TB), lambda b, t: (b, 0, t)),
            pl.BlockSpec((1, 1, 128), lambda b, t: (b, 0, 0)),
        ],
        out_shape=[
            jax.ShapeDtypeStruct((B, D, T), jnp.float32),
            jax.ShapeDtypeStruct((B, 1, 128), jnp.float32),
        ],
        compiler_params=pltpu.CompilerParams(
            dimension_semantics=("arbitrary", "arbitrary")),
    )(x, cba, cbt4)
    losses = lossv[:, 0, 0] * (1.0 / (D * T))
    return (out, losses)
